# baseline (speedup 1.0000x reference)
import numpy as np
import jax
import jax.numpy as jnp
from jax import lax
from jax.experimental import pallas as pl
from jax.experimental.pallas import tpu as pltpu
from jax.experimental.pallas import tpu_sc as plsc

D_MODEL = 64
MAX_SEQ_LEN = 200
BATCH = 4096

_NC = 2
_NS = 16
_NW = _NC * _NS
_ROWS_PER_W = BATCH // _NW
_L = 16
_G = 8
_NGRP = MAX_SEQ_LEN // _G
_GBITS = (16, 8, 4, 2, 1)
_ZGRP = 16
_ROWW_E = MAX_SEQ_LEN - _G


def _pe_body(len_hbm, tab2_hbm, emb_hbm, pos_hbm,
             len_v, tab_v, zero_v, pid_v, tail_v, sem_e, sem_t, sem_p):
    wid = lax.axis_index("s") * _NC + lax.axis_index("c")
    row_base = pl.multiple_of(wid * _ROWS_PER_W, _ROWS_PER_W)
    pltpu.sync_copy(len_hbm.at[pl.ds(row_base, _ROWS_PER_W)],
                    len_v.at[pl.ds(0, _ROWS_PER_W)])
    pltpu.sync_copy(tab2_hbm, tab_v)

    zvec = jnp.zeros((_L,), jnp.float32)

    def zfill(i, carry):
        r = i // (D_MODEL // _L)
        c = i - r * (D_MODEL // _L)
        zero_v[r, pl.ds(c * _L, _L)] = zvec
        return carry

    lax.fori_loop(0, _ZGRP * _G * D_MODEL // _L, zfill, 0)

    def row(r, carry):
        ln = len_v[pl.ds(r, _L)][0]
        q = lax.shift_right_logical(ln, 3)
        m = ln & 7
        out0 = (row_base + r) * MAX_SEQ_LEN
        off = jnp.int32(0)
        for gbit in _GBITS:
            @pl.when((q & gbit) != 0)
            def _(off=off, gbit=gbit):
                pltpu.async_copy(
                    tab_v.at[pl.ds(pl.multiple_of(off, _G), gbit * _G)],
                    emb_hbm.at[pl.ds(pl.multiple_of(out0 + off, _G),
                                     gbit * _G)], sem_e)
            off = off + (q & gbit) * _G
        slot = r & 7
        @pl.when(r >= 8)
        def _():
            pltpu.make_async_copy(
                emb_hbm.at[pl.ds(0, _G)],
                tail_v.at[pl.ds(0, _G)], sem_t).wait()
        s0 = pl.multiple_of(slot * _G, _G)
        mf = m.astype(jnp.float32)
        for j in range(_G):
            factor = jnp.where(mf > j, jnp.float32(1.0), jnp.float32(0.0))
            trow = q * _G + j
            for c in range(D_MODEL // _L):
                tail_v[s0 + j, pl.ds(c * _L, _L)] = (
                    tab_v[trow, pl.ds(c * _L, _L)] * factor)
        pltpu.async_copy(
            tail_v.at[pl.ds(s0, _G)],
            emb_hbm.at[pl.ds(pl.multiple_of(out0 + off, _G), _G)], sem_t)
        off = off + _G
        zq = (_NGRP - 1) - q
        for gbit in _GBITS:
            @pl.when((zq & gbit) != 0)
            def _(off=off, gbit=gbit):
                pltpu.async_copy(
                    zero_v.at[pl.ds(0, gbit * _G)],
                    emb_hbm.at[pl.ds(pl.multiple_of(out0 + off, _G),
                                     gbit * _G)], sem_e)
            off = off + (zq & gbit) * _G

        lenv = plsc.load_gather(len_v, [jnp.full((_L,), r, jnp.int32)])
        for s in (*range(0, 192, 16), 184):
            pos = s + 1 + lax.iota(jnp.int32, _L)
            pid_v[r, pl.ds(s, _L)] = jnp.where(pos <= lenv, pos, 0)
        return carry

    lax.fori_loop(0, _ROWS_PER_W, row, 0)
    ph = pltpu.async_copy(pid_v, pos_hbm.at[pl.ds(row_base, _ROWS_PER_W)],
                          sem_p)

    def drain(r, carry):
        pltpu.make_async_copy(
            emb_hbm.at[pl.ds(0, _ROWW_E)],
            tab_v.at[pl.ds(0, _ROWW_E)], sem_e).wait()
        return carry

    lax.fori_loop(0, _ROWS_PER_W, drain, 0)

    def drain_t(r, carry):
        pltpu.make_async_copy(
            emb_hbm.at[pl.ds(0, _G)],
            tail_v.at[pl.ds(0, _G)], sem_t).wait()
        return carry

    lax.fori_loop(0, 8, drain_t, 0)
    ph.wait()


def kernel(input_len, device, table):
    del device
    tab2 = table[1:]
    mesh = plsc.VectorSubcoreMesh(core_axis_name="c", subcore_axis_name="s")
    k = pl.kernel(
        _pe_body,
        mesh=mesh,
        compiler_params=pltpu.CompilerParams(needs_layout_passes=False),
        out_type=[
            jax.ShapeDtypeStruct((BATCH, MAX_SEQ_LEN), jnp.int32),
        ],
        scratch_types=[
            pltpu.VMEM((_ROWS_PER_W + _L,), jnp.int32),
            pltpu.VMEM((MAX_SEQ_LEN, D_MODEL), jnp.float32),
            pltpu.VMEM((_ZGRP * _G, D_MODEL), jnp.float32),
            pltpu.VMEM((_ROWS_PER_W, MAX_SEQ_LEN), jnp.int32),
            pltpu.VMEM((8 * _G, D_MODEL), jnp.float32),
            pltpu.SemaphoreType.DMA,
            pltpu.SemaphoreType.DMA,
            pltpu.SemaphoreType.DMA,
        ],
    )
    emb_ref = jax.new_ref(
        jnp.zeros((BATCH * MAX_SEQ_LEN, D_MODEL), jnp.float32))
    (pos,) = k(input_len.astype(jnp.int32), tab2, emb_ref)
    return (emb_ref[...].reshape(BATCH, MAX_SEQ_LEN, D_MODEL), pos)


# speedup vs baseline: 1.3745x; 1.3745x over previous
import jax
import jax.numpy as jnp
from jax import lax
from jax.experimental import pallas as pl
from jax.experimental.pallas import tpu as pltpu
from jax.experimental.pallas import tpu_sc as plsc

D_MODEL = 64
MAX_SEQ_LEN = 200
BATCH = 4096

_NC = 2
_NS = 16
_NW = _NC * _NS
_ROWS_PER_W = BATCH // _NW
_L = 16
_G = 8
_NGRP = MAX_SEQ_LEN // _G
_GBITS = (16, 8, 4, 2, 1)
_ZGRP = 16
_ROWW_E = MAX_SEQ_LEN - _G


def _pe_body(len_hbm, tab2_hbm, emb_hbm, pos_hbm,
             len_v, tab_v, zero_v, pid_v, tail_v, sem_e, sem_t, sem_p):
    wid = lax.axis_index("s") * _NC + lax.axis_index("c")
    row_base = pl.multiple_of(wid * _ROWS_PER_W, _ROWS_PER_W)
    pltpu.sync_copy(len_hbm.at[pl.ds(row_base, _ROWS_PER_W)],
                    len_v.at[pl.ds(0, _ROWS_PER_W)])
    pltpu.sync_copy(tab2_hbm, tab_v)

    zvec = jnp.zeros((_L,), jnp.float32)

    def zfill(i, carry):
        r = i // (D_MODEL // _L)
        c = i - r * (D_MODEL // _L)
        zero_v[r, pl.ds(c * _L, _L)] = zvec
        return carry

    lax.fori_loop(0, _ZGRP * _G * D_MODEL // _L, zfill, 0)

    def row(r, carry):
        ln = len_v[pl.ds(r, _L)][0]
        q = lax.shift_right_logical(ln, 3)
        m = ln & 7
        out0 = (row_base + r) * MAX_SEQ_LEN
        off = jnp.int32(0)
        for gbit in _GBITS:
            @pl.when((q & gbit) != 0)
            def _(off=off, gbit=gbit):
                pltpu.async_copy(
                    tab_v.at[pl.ds(pl.multiple_of(off, _G), gbit * _G)],
                    emb_hbm.at[pl.ds(pl.multiple_of(out0 + off, _G),
                                     gbit * _G)], sem_e)
            off = off + (q & gbit) * _G
        slot = r & 7
        @pl.when(r >= 8)
        def _():
            pltpu.make_async_copy(
                emb_hbm.at[pl.ds(0, _G)],
                tail_v.at[pl.ds(0, _G)], sem_t).wait()
        s0 = pl.multiple_of(slot * _G, _G)
        mf = m.astype(jnp.float32)
        for j in range(_G):
            factor = jnp.where(mf > j, jnp.float32(1.0), jnp.float32(0.0))
            trow = q * _G + j
            for c in range(D_MODEL // _L):
                tail_v[s0 + j, pl.ds(c * _L, _L)] = (
                    tab_v[trow, pl.ds(c * _L, _L)] * factor)
        pltpu.async_copy(
            tail_v.at[pl.ds(s0, _G)],
            emb_hbm.at[pl.ds(pl.multiple_of(out0 + off, _G), _G)], sem_t)
        off = off + _G
        zq = (_NGRP - 1) - q
        for gbit in _GBITS:
            @pl.when((zq & gbit) != 0)
            def _(off=off, gbit=gbit):
                pltpu.async_copy(
                    zero_v.at[pl.ds(0, gbit * _G)],
                    emb_hbm.at[pl.ds(pl.multiple_of(out0 + off, _G),
                                     gbit * _G)], sem_e)
            off = off + (zq & gbit) * _G

        lenv = plsc.load_gather(len_v, [jnp.full((_L,), r, jnp.int32)])
        for s in (*range(0, 192, 16), 184):
            pos = s + 1 + lax.iota(jnp.int32, _L)
            pid_v[r, pl.ds(s, _L)] = jnp.where(pos <= lenv, pos, 0)
        return carry

    lax.fori_loop(0, _ROWS_PER_W, row, 0)
    ph = pltpu.async_copy(pid_v, pos_hbm.at[pl.ds(row_base, _ROWS_PER_W)],
                          sem_p)

    def drain(r, carry):
        pltpu.make_async_copy(
            emb_hbm.at[pl.ds(0, _ROWW_E)],
            tab_v.at[pl.ds(0, _ROWW_E)], sem_e).wait()
        return carry

    lax.fori_loop(0, _ROWS_PER_W, drain, 0)

    def drain_t(r, carry):
        pltpu.make_async_copy(
            emb_hbm.at[pl.ds(0, _G)],
            tail_v.at[pl.ds(0, _G)], sem_t).wait()
        return carry

    lax.fori_loop(0, 8, drain_t, 0)
    ph.wait()


def kernel(input_len, device, table):
    del device
    tab2 = table[1:]
    mesh = plsc.VectorSubcoreMesh(core_axis_name="c", subcore_axis_name="s")
    k = pl.kernel(
        _pe_body,
        mesh=mesh,
        compiler_params=pltpu.CompilerParams(needs_layout_passes=False),
        out_type=[
            jax.ShapeDtypeStruct((BATCH * MAX_SEQ_LEN, D_MODEL),
                                 jnp.float32),
            jax.ShapeDtypeStruct((BATCH, MAX_SEQ_LEN), jnp.int32),
        ],
        scratch_types=[
            pltpu.VMEM((_ROWS_PER_W + _L,), jnp.int32),
            pltpu.VMEM((MAX_SEQ_LEN, D_MODEL), jnp.float32),
            pltpu.VMEM((_ZGRP * _G, D_MODEL), jnp.float32),
            pltpu.VMEM((_ROWS_PER_W, MAX_SEQ_LEN), jnp.int32),
            pltpu.VMEM((8 * _G, D_MODEL), jnp.float32),
            pltpu.SemaphoreType.DMA,
            pltpu.SemaphoreType.DMA,
            pltpu.SemaphoreType.DMA,
        ],
    )
    emb_flat, pos = k(input_len.astype(jnp.int32), tab2)
    return (emb_flat.reshape(BATCH, MAX_SEQ_LEN, D_MODEL), pos)
